# initial kernel scaffold (unmeasured)
import jax
import jax.numpy as jnp
from jax import lax
from jax.experimental import pallas as pl
from jax.experimental.pallas import tpu as pltpu

Y = 4


def kernel(x, W):
    t, d = x.shape
    _, v_loc = W.shape
    v = Y * v_loc

    def body(x_ref, w_ref, out_ref, send_sems, recv_sems):
        my_x = lax.axis_index("x")
        my_y = lax.axis_index("y")
        my_z = lax.axis_index("z")
        left = lax.rem(my_y + (Y - 1), Y)
        right = lax.rem(my_y + 1, Y)

        barrier = pltpu.get_barrier_semaphore()
        for nbr in (left, right):
            pl.semaphore_signal(
                barrier,
                inc=1,
                device_id=(my_x, nbr, my_z),
                device_id_type=pl.DeviceIdType.MESH,
            )
        pl.semaphore_wait(barrier, 2)

        logits = jnp.dot(
            x_ref[...], w_ref[...], preferred_element_type=jnp.float32
        )
        for c in range(Y):

            @pl.when(my_y == c)
            def _(c=c):
                out_ref[:, c * v_loc : (c + 1) * v_loc] = logits

        for h in range(Y - 1):
            for c in range(Y):
                origin = (c - h) % Y
                sl = slice(origin * v_loc, (origin + 1) * v_loc)

                @pl.when(my_y == c)
                def _(sl=sl):
                    rdma = pltpu.make_async_remote_copy(
                        src_ref=out_ref.at[:, sl],
                        dst_ref=out_ref.at[:, sl],
                        send_sem=send_sems.at[h],
                        recv_sem=recv_sems.at[h],
                        device_id=(my_x, right, my_z),
                        device_id_type=pl.DeviceIdType.MESH,
                    )
                    rdma.start()
                    rdma.wait()

        m = jnp.full((t, 1), -jnp.inf, dtype=jnp.float32)
        for c in range(Y):
            blk = out_ref[:, c * v_loc : (c + 1) * v_loc]
            m = jnp.maximum(m, jnp.max(blk, axis=1, keepdims=True))
        s = jnp.zeros((t, 1), dtype=jnp.float32)
        for c in range(Y):
            e = jnp.exp(out_ref[:, c * v_loc : (c + 1) * v_loc] - m)
            out_ref[:, c * v_loc : (c + 1) * v_loc] = e
            s = s + jnp.sum(e, axis=1, keepdims=True)
        r = 1.0 / s
        for c in range(Y):
            out_ref[:, c * v_loc : (c + 1) * v_loc] = (
                out_ref[:, c * v_loc : (c + 1) * v_loc] * r
            )

    return pl.pallas_call(
        body,
        out_shape=jax.ShapeDtypeStruct((t, v), jnp.float32),
        in_specs=[
            pl.BlockSpec(memory_space=pltpu.VMEM),
            pl.BlockSpec(memory_space=pltpu.VMEM),
        ],
        out_specs=pl.BlockSpec(memory_space=pltpu.VMEM),
        scratch_shapes=[
            pltpu.SemaphoreType.DMA((Y - 1,)),
            pltpu.SemaphoreType.DMA((Y - 1,)),
        ],
        compiler_params=pltpu.CompilerParams(collective_id=0),
    )(x, W)


# baseline (device time: 750188 ns/iter reference)
import jax
import jax.numpy as jnp
from jax import lax
from jax.experimental import pallas as pl
from jax.experimental.pallas import tpu as pltpu

Y = 4
SUB = 2048


def kernel(x, W):
    t, d = x.shape
    _, v_loc = W.shape
    v = Y * v_loc
    nsub = v_loc // SUB

    def body(x_ref, w_ref, out_ref, tile, w_tile, dma_sem, w_sem, send_sems, recv_sems):
        my_x = lax.axis_index("x")
        my_y = lax.axis_index("y")
        my_z = lax.axis_index("z")
        left = lax.rem(my_y + (Y - 1), Y)
        right = lax.rem(my_y + 1, Y)

        barrier = pltpu.get_barrier_semaphore()
        for nbr in (left, right):
            pl.semaphore_signal(
                barrier,
                inc=1,
                device_id=(my_x, nbr, my_z),
                device_id_type=pl.DeviceIdType.MESH,
            )
        pl.semaphore_wait(barrier, 2)

        def gemm_step(j, carry):
            wcp = pltpu.make_async_copy(
                w_ref.at[:, pl.ds(j * SUB, SUB)], w_tile, w_sem
            )
            wcp.start()
            wcp.wait()
            tile[...] = jnp.dot(
                x_ref[...], w_tile[...], preferred_element_type=jnp.float32
            )
            cp = pltpu.make_async_copy(
                tile,
                out_ref.at[:, pl.ds(my_y * v_loc + j * SUB, SUB)],
                dma_sem,
            )
            cp.start()
            cp.wait()
            return carry

        lax.fori_loop(0, nsub, gemm_step, 0)

        for h in range(Y - 1):
            origin = lax.rem(my_y + (Y - h), Y)
            rdma = pltpu.make_async_remote_copy(
                src_ref=out_ref.at[:, pl.ds(origin * v_loc, v_loc)],
                dst_ref=out_ref.at[:, pl.ds(origin * v_loc, v_loc)],
                send_sem=send_sems.at[h],
                recv_sem=recv_sems.at[h],
                device_id=(my_x, right, my_z),
                device_id_type=pl.DeviceIdType.MESH,
            )
            rdma.start()
            rdma.wait()

        def p1(k, carry):
            m, s = carry
            cp = pltpu.make_async_copy(
                out_ref.at[:, pl.ds(k * SUB, SUB)], tile, dma_sem
            )
            cp.start()
            cp.wait()
            blk = tile[...]
            m_new = jnp.maximum(m, jnp.max(blk, axis=1, keepdims=True))
            s = s * jnp.exp(m - m_new) + jnp.sum(
                jnp.exp(blk - m_new), axis=1, keepdims=True
            )
            return (m_new, s)

        m0 = jnp.full((t, 1), -jnp.inf, dtype=jnp.float32)
        s0 = jnp.zeros((t, 1), dtype=jnp.float32)
        m, s = lax.fori_loop(0, Y * nsub, p1, (m0, s0))
        r = 1.0 / s

        def p2(k, carry):
            cp = pltpu.make_async_copy(
                out_ref.at[:, pl.ds(k * SUB, SUB)], tile, dma_sem
            )
            cp.start()
            cp.wait()
            tile[...] = jnp.exp(tile[...] - m) * r
            cp2 = pltpu.make_async_copy(
                tile, out_ref.at[:, pl.ds(k * SUB, SUB)], dma_sem
            )
            cp2.start()
            cp2.wait()
            return carry

        lax.fori_loop(0, Y * nsub, p2, 0)

    return pl.pallas_call(
        body,
        out_shape=jax.ShapeDtypeStruct((t, v), jnp.float32),
        in_specs=[
            pl.BlockSpec(memory_space=pltpu.VMEM),
            pl.BlockSpec(memory_space=pl.ANY),
        ],
        out_specs=pl.BlockSpec(memory_space=pl.ANY),
        scratch_shapes=[
            pltpu.VMEM((t, SUB), jnp.float32),
            pltpu.VMEM((d, SUB), jnp.float32),
            pltpu.SemaphoreType.DMA,
            pltpu.SemaphoreType.DMA,
            pltpu.SemaphoreType.DMA((Y - 1,)),
            pltpu.SemaphoreType.DMA((Y - 1,)),
        ],
        compiler_params=pltpu.CompilerParams(collective_id=0),
    )(x, W)


# device time: 478779 ns/iter; 1.5669x vs baseline; 1.5669x over previous
import jax
import jax.numpy as jnp
from jax import lax
from jax.experimental import pallas as pl
from jax.experimental.pallas import tpu as pltpu

Y = 4
SUB = 2048
NHOP = Y - 1


def kernel(x, W):
    t, d = x.shape
    _, v_loc = W.shape
    v = Y * v_loc
    nsub = v_loc // SUB
    h_rows = t // 2

    def body(x_ref, w_ref, out_ref, tile, w_tile, dma_sem, w_sem,
             y_send, y_recv, x_send, x_recv):
        my_x = lax.axis_index("x")
        my_y = lax.axis_index("y")
        my_z = lax.axis_index("z")
        left = lax.rem(my_y + (Y - 1), Y)
        right = lax.rem(my_y + 1, Y)
        rx = my_x * h_rows

        barrier = pltpu.get_barrier_semaphore()
        for dev in ((my_x, left, my_z), (my_x, right, my_z),
                    (1 - my_x, my_y, my_z)):
            pl.semaphore_signal(
                barrier, inc=1, device_id=dev,
                device_id_type=pl.DeviceIdType.MESH,
            )
        pl.semaphore_wait(barrier, 3)

        def half_slab(chunk, j):
            return out_ref.at[
                pl.ds(rx, h_rows), pl.ds(chunk * v_loc + j * SUB, SUB)
            ]

        y_rdmas = {}
        x_rdmas = {}

        for j in range(nsub):
            wcp = pltpu.make_async_copy(
                w_ref.at[:, pl.ds(j * SUB, SUB)], w_tile, w_sem
            )
            wcp.start()
            wcp.wait()
            tile[...] = jnp.dot(
                x_ref[...], w_tile[...], preferred_element_type=jnp.float32
            )
            cp = pltpu.make_async_copy(
                tile,
                out_ref.at[:, pl.ds(my_y * v_loc + j * SUB, SUB)],
                dma_sem,
            )
            cp.start()
            cp.wait()
            rdma = pltpu.make_async_remote_copy(
                src_ref=half_slab(my_y, j),
                dst_ref=half_slab(my_y, j),
                send_sem=y_send.at[0, j],
                recv_sem=y_recv.at[0, j],
                device_id=(my_x, right, my_z),
                device_id_type=pl.DeviceIdType.MESH,
            )
            rdma.start()
            y_rdmas[(0, j)] = rdma

        for h in range(NHOP):
            got = lax.rem(my_y + (Y - h - 1), Y)
            for j in range(nsub):
                y_rdmas[(h, j)].wait_recv()
                if h + 1 < NHOP:
                    rdma = pltpu.make_async_remote_copy(
                        src_ref=half_slab(got, j),
                        dst_ref=half_slab(got, j),
                        send_sem=y_send.at[h + 1, j],
                        recv_sem=y_recv.at[h + 1, j],
                        device_id=(my_x, right, my_z),
                        device_id_type=pl.DeviceIdType.MESH,
                    )
                    rdma.start()
                    y_rdmas[(h + 1, j)] = rdma
                xr = pltpu.make_async_remote_copy(
                    src_ref=half_slab(got, j),
                    dst_ref=half_slab(got, j),
                    send_sem=x_send.at[h, j],
                    recv_sem=x_recv.at[h, j],
                    device_id=(1 - my_x, my_y, my_z),
                    device_id_type=pl.DeviceIdType.MESH,
                )
                xr.start()
                x_rdmas[(h, j)] = xr

        for h in range(NHOP):
            for j in range(nsub):
                x_rdmas[(h, j)].wait_recv()
        for h in range(NHOP):
            for j in range(nsub):
                y_rdmas[(h, j)].wait_send()
                x_rdmas[(h, j)].wait_send()

        def p1(k, carry):
            m, s = carry
            cp = pltpu.make_async_copy(
                out_ref.at[:, pl.ds(k * SUB, SUB)], tile, dma_sem
            )
            cp.start()
            cp.wait()
            blk = tile[...]
            m_new = jnp.maximum(m, jnp.max(blk, axis=1, keepdims=True))
            s = s * jnp.exp(m - m_new) + jnp.sum(
                jnp.exp(blk - m_new), axis=1, keepdims=True
            )
            return (m_new, s)

        m0 = jnp.full((t, 1), -jnp.inf, dtype=jnp.float32)
        s0 = jnp.zeros((t, 1), dtype=jnp.float32)
        m, s = lax.fori_loop(0, Y * nsub, p1, (m0, s0))
        r = 1.0 / s

        def p2(k, carry):
            cp = pltpu.make_async_copy(
                out_ref.at[:, pl.ds(k * SUB, SUB)], tile, dma_sem
            )
            cp.start()
            cp.wait()
            tile[...] = jnp.exp(tile[...] - m) * r
            cp2 = pltpu.make_async_copy(
                tile, out_ref.at[:, pl.ds(k * SUB, SUB)], dma_sem
            )
            cp2.start()
            cp2.wait()
            return carry

        lax.fori_loop(0, Y * nsub, p2, 0)

    return pl.pallas_call(
        body,
        out_shape=jax.ShapeDtypeStruct((t, v), jnp.float32),
        in_specs=[
            pl.BlockSpec(memory_space=pltpu.VMEM),
            pl.BlockSpec(memory_space=pl.ANY),
        ],
        out_specs=pl.BlockSpec(memory_space=pl.ANY),
        scratch_shapes=[
            pltpu.VMEM((t, SUB), jnp.float32),
            pltpu.VMEM((d, SUB), jnp.float32),
            pltpu.SemaphoreType.DMA,
            pltpu.SemaphoreType.DMA,
            pltpu.SemaphoreType.DMA((NHOP, v_loc // SUB)),
            pltpu.SemaphoreType.DMA((NHOP, v_loc // SUB)),
            pltpu.SemaphoreType.DMA((NHOP, v_loc // SUB)),
            pltpu.SemaphoreType.DMA((NHOP, v_loc // SUB)),
        ],
        compiler_params=pltpu.CompilerParams(collective_id=0),
    )(x, W)


# device time: 417665 ns/iter; 1.7961x vs baseline; 1.1463x over previous
import jax
import jax.numpy as jnp
from jax import lax
from jax.experimental import pallas as pl
from jax.experimental.pallas import tpu as pltpu

Y = 4
SUB = 2048
NHOP = Y - 1


def kernel(x, W):
    t, d = x.shape
    _, v_loc = W.shape
    v = Y * v_loc
    nsub = v_loc // SUB
    h_rows = t // 2

    def body(x_ref, w_ref, out_ref, tile, tile2, w_tile, stat_tile,
             stats, dma_sem, w_sem, io_sems, st_sems, y_send, y_recv,
             x_send, x_recv):
        my_x = lax.axis_index("x")
        my_y = lax.axis_index("y")
        my_z = lax.axis_index("z")
        left = lax.rem(my_y + (Y - 1), Y)
        right = lax.rem(my_y + 1, Y)
        rx = my_x * h_rows

        barrier = pltpu.get_barrier_semaphore()
        for dev in ((my_x, left, my_z), (my_x, right, my_z),
                    (1 - my_x, my_y, my_z)):
            pl.semaphore_signal(
                barrier, inc=1, device_id=dev,
                device_id_type=pl.DeviceIdType.MESH,
            )
        pl.semaphore_wait(barrier, 3)

        def half_slab(chunk, j):
            return out_ref.at[
                pl.ds(rx, h_rows), pl.ds(chunk * v_loc + j * SUB, SUB)
            ]

        def upd(carry, blk):
            mH, sH = carry
            m_new = jnp.maximum(mH, jnp.max(blk, axis=1, keepdims=True))
            sH = sH * jnp.exp(mH - m_new) + jnp.sum(
                jnp.exp(blk - m_new), axis=1, keepdims=True
            )
            return (m_new, sH)

        st = (
            jnp.full((h_rows, 1), -jnp.inf, dtype=jnp.float32),
            jnp.zeros((h_rows, 1), dtype=jnp.float32),
        )

        y_rdmas = {}
        x_rdmas = {}

        for j in range(nsub):
            wcp = pltpu.make_async_copy(
                w_ref.at[:, pl.ds(j * SUB, SUB)], w_tile, w_sem
            )
            wcp.start()
            wcp.wait()
            tile[...] = jnp.dot(
                x_ref[...], w_tile[...], preferred_element_type=jnp.float32
            )
            cp = pltpu.make_async_copy(
                tile,
                out_ref.at[:, pl.ds(my_y * v_loc + j * SUB, SUB)],
                dma_sem,
            )
            cp.start()
            cp.wait()
            rdma = pltpu.make_async_remote_copy(
                src_ref=half_slab(my_y, j),
                dst_ref=half_slab(my_y, j),
                send_sem=y_send.at[0, j],
                recv_sem=y_recv.at[0, j],
                device_id=(my_x, right, my_z),
                device_id_type=pl.DeviceIdType.MESH,
            )
            rdma.start()
            y_rdmas[(0, j)] = rdma
            st = upd(st, tile[pl.ds(rx, h_rows), :])

        for h in range(NHOP):
            got = lax.rem(my_y + (Y - h - 1), Y)
            for j in range(nsub):
                y_rdmas[(h, j)].wait_recv()
                if h + 1 < NHOP:
                    rdma = pltpu.make_async_remote_copy(
                        src_ref=half_slab(got, j),
                        dst_ref=half_slab(got, j),
                        send_sem=y_send.at[h + 1, j],
                        recv_sem=y_recv.at[h + 1, j],
                        device_id=(my_x, right, my_z),
                        device_id_type=pl.DeviceIdType.MESH,
                    )
                    rdma.start()
                    y_rdmas[(h + 1, j)] = rdma
                xr = pltpu.make_async_remote_copy(
                    src_ref=half_slab(got, j),
                    dst_ref=half_slab(got, j),
                    send_sem=x_send.at[h, j],
                    recv_sem=x_recv.at[h, j],
                    device_id=(1 - my_x, my_y, my_z),
                    device_id_type=pl.DeviceIdType.MESH,
                )
                xr.start()
                x_rdmas[(h, j)] = xr
                scp = pltpu.make_async_copy(
                    half_slab(got, j), stat_tile, dma_sem
                )
                scp.start()
                scp.wait()
                st = upd(st, stat_tile[...])

        mH, sH = st
        stats[0, pl.ds(rx, h_rows), :] = jnp.broadcast_to(mH, (h_rows, 128))
        stats[1, pl.ds(rx, h_rows), :] = jnp.broadcast_to(sH, (h_rows, 128))
        str_ = pltpu.make_async_remote_copy(
            src_ref=stats.at[:, pl.ds(rx, h_rows), :],
            dst_ref=stats.at[:, pl.ds(rx, h_rows), :],
            send_sem=st_sems.at[0],
            recv_sem=st_sems.at[1],
            device_id=(1 - my_x, my_y, my_z),
            device_id_type=pl.DeviceIdType.MESH,
        )
        str_.start()
        str_.wait()

        for h in range(NHOP):
            for j in range(nsub):
                x_rdmas[(h, j)].wait_recv()
        for h in range(NHOP):
            for j in range(nsub):
                y_rdmas[(h, j)].wait_send()
                x_rdmas[(h, j)].wait_send()

        m = stats[0, :, 0:1]
        r = 1.0 / stats[1, :, 0:1]

        n_blk = Y * nsub
        bufs = (tile, tile2)
        loads = {}
        stores = {}

        def load(k, buf):
            cp = pltpu.make_async_copy(
                out_ref.at[:, pl.ds(k * SUB, SUB)], buf, io_sems.at[k % 2]
            )
            cp.start()
            return cp

        loads[0] = load(0, bufs[0])
        for k in range(n_blk):
            b = bufs[k % 2]
            loads[k].wait()
            if k + 1 < n_blk:
                if k - 1 >= 0:
                    stores[k - 1].wait()
                loads[k + 1] = load(k + 1, bufs[(k + 1) % 2])
            b[...] = jnp.exp(b[...] - m) * r
            cp = pltpu.make_async_copy(
                b, out_ref.at[:, pl.ds(k * SUB, SUB)], io_sems.at[2 + k % 2]
            )
            cp.start()
            stores[k] = cp
        stores[n_blk - 2].wait()
        stores[n_blk - 1].wait()

    return pl.pallas_call(
        body,
        out_shape=jax.ShapeDtypeStruct((t, v), jnp.float32),
        in_specs=[
            pl.BlockSpec(memory_space=pltpu.VMEM),
            pl.BlockSpec(memory_space=pl.ANY),
        ],
        out_specs=pl.BlockSpec(memory_space=pl.ANY),
        scratch_shapes=[
            pltpu.VMEM((t, SUB), jnp.float32),
            pltpu.VMEM((t, SUB), jnp.float32),
            pltpu.VMEM((d, SUB), jnp.float32),
            pltpu.VMEM((h_rows, SUB), jnp.float32),
            pltpu.VMEM((2, t, 128), jnp.float32),
            pltpu.SemaphoreType.DMA,
            pltpu.SemaphoreType.DMA,
            pltpu.SemaphoreType.DMA((4,)),
            pltpu.SemaphoreType.DMA((2,)),
            pltpu.SemaphoreType.DMA((NHOP, v_loc // SUB)),
            pltpu.SemaphoreType.DMA((NHOP, v_loc // SUB)),
            pltpu.SemaphoreType.DMA((NHOP, v_loc // SUB)),
            pltpu.SemaphoreType.DMA((NHOP, v_loc // SUB)),
        ],
        compiler_params=pltpu.CompilerParams(collective_id=0),
    )(x, W)
